# baseline (speedup 1.0000x reference)
import functools

import jax
import jax.numpy as jnp
from jax import lax
from jax.experimental import pallas as pl
from jax.experimental.pallas import tpu as pltpu
from jax.experimental.pallas import tpu_sc as plsc

N_UP = 2500
F = 128
K = 32
N_PAD = 10240
NW = 32
RPW = N_PAD // NW
C = 16
NCHUNK = RPW // C
L = 16


def _body(feat_hbm, w_hbm, nidx_hbm, out_hbm, idx2d, w2d, g_v, out_buf, sem):
    nc = 2
    wid = lax.axis_index("s") * nc + lax.axis_index("c")
    row0_w = wid * RPW

    def chunk(ci, carry):
        row0 = row0_w + ci * C
        pltpu.sync_copy(nidx_hbm.at[pl.ds(row0, C)], idx2d)
        pltpu.sync_copy(w_hbm.at[pl.ds(row0, C)], w2d)
        cps = []
        for r in range(C):
            for h in range(K // L):
                v = idx2d[r, pl.ds(h * L, L)]
                m = v < N_UP
                vm = jnp.where(m, v, 0)
                wv = w2d[r, pl.ds(h * L, L)]
                w2d[r, pl.ds(h * L, L)] = jnp.where(m, wv, 0.0)
                cps.append(
                    pltpu.async_copy(feat_hbm.at[vm],
                                     g_v.at[pl.ds((r * 2 + h) * L, L)], sem))
        for cp in cps:
            cp.wait()

        def row_body(r, _):
            accs = tuple(jnp.zeros((L,), jnp.float32) for _ in range(F // L))
            for g in range(K // L):
                wv = w2d[r, pl.ds(g * L, L)]
                for lane in range(L):
                    ws = wv[lane]
                    gb = r * K + g * L + lane
                    accs = tuple(accs[h] + ws * g_v[gb, pl.ds(h * L, L)]
                                 for h in range(F // L))
            for h in range(F // L):
                out_buf[r, pl.ds(h * L, L)] = accs[h] * (1.0 / K)
            return 0

        lax.fori_loop(0, C, row_body, 0)
        pltpu.sync_copy(out_buf, out_hbm.at[pl.ds(row0, C)])
        return carry

    lax.fori_loop(0, NCHUNK, chunk, 0)


@jax.jit
def _sc_call(features, w_p, n_p):
    mesh = plsc.VectorSubcoreMesh(core_axis_name="c", subcore_axis_name="s")
    return pl.kernel(
        _body,
        out_type=jax.ShapeDtypeStruct((N_PAD, F), jnp.float32),
        mesh=mesh,
        scratch_types=[
            pltpu.VMEM((C, K), jnp.int32),
            pltpu.VMEM((C, K), jnp.float32),
            pltpu.VMEM((C * K, F), jnp.float32),
            pltpu.VMEM((C, F), jnp.float32),
            pltpu.SemaphoreType.DMA,
        ],
    )(features, w_p, n_p)


def kernel(features, sel_idx_up, weights_down, nidx_down):
    n_down = weights_down.shape[0]
    pad = N_PAD - n_down
    w_p = jnp.pad(weights_down, ((0, pad), (0, 0)))
    n_p = jnp.pad(nidx_down, ((0, pad), (0, 0)))
    out = _sc_call(features, w_p, n_p)
    return out[:n_down]


# speedup vs baseline: 63.5179x; 63.5179x over previous
import functools

import jax
import jax.numpy as jnp
from jax import lax
from jax.experimental import pallas as pl
from jax.experimental.pallas import tpu as pltpu
from jax.experimental.pallas import tpu_sc as plsc

N_UP = 2500
F = 128
K = 32
N_PAD = 10240
L = 16

RG = 8
FC = 4
RPG = N_PAD // RG
SUB = 5
RPS = RPG // SUB
NBLK = RPS // L
FCW = F // FC


def _body(feat_hbm, wt_hbm, nt_hbm, out_hbm, feat_c, idx_c, w_c, out_buf):
    wid = lax.axis_index("s") * 2 + lax.axis_index("c")
    rg = wid // FC
    fc = wid % FC
    row0g = rg * RPG
    col0 = fc * FCW
    pltpu.sync_copy(feat_hbm.at[pl.ds(col0, FCW)], feat_c)

    def sub(s, _):
        row0 = row0g + s * RPS
        pltpu.sync_copy(nt_hbm.at[:, pl.ds(row0, RPS)], idx_c)
        pltpu.sync_copy(wt_hbm.at[:, pl.ds(row0, RPS)], w_c)

        def block(b, _):
            rr = b * L
            for half in range(2):
                def kbody(k, accs):
                    vk = idx_c[k, pl.ds(rr, L)]
                    m = vk < N_UP
                    vkc = jnp.where(m, vk, 0)
                    wk = jnp.where(m, w_c[k, pl.ds(rr, L)], 0.0)
                    new = []
                    for f in range(L):
                        col = jnp.full((L,), half * L + f, jnp.int32)
                        g = plsc.load_gather(feat_c, [col, vkc])
                        new.append(accs[f] + wk * g)
                    return tuple(new)

                accs = lax.fori_loop(
                    0, K, kbody,
                    tuple(jnp.zeros((L,), jnp.float32) for _ in range(L)))
                for f in range(L):
                    out_buf[half * L + f, pl.ds(rr, L)] = accs[f] * (1.0 / K)
            return 0

        lax.fori_loop(0, NBLK, block, 0)
        pltpu.sync_copy(out_buf,
                        out_hbm.at[pl.ds(col0, FCW), pl.ds(row0, RPS)])
        return 0

    lax.fori_loop(0, SUB, sub, 0)


@jax.jit
def _sc_call(feat_t, wt, nt):
    mesh = plsc.VectorSubcoreMesh(core_axis_name="c", subcore_axis_name="s")
    return pl.kernel(
        _body,
        out_type=jax.ShapeDtypeStruct((F, N_PAD), jnp.float32),
        mesh=mesh,
        compiler_params=pltpu.CompilerParams(use_tc_tiling_on_sc=False,
                                             needs_layout_passes=False),
        scratch_types=[
            pltpu.VMEM((FCW, N_UP), jnp.float32),
            pltpu.VMEM((K, RPS), jnp.int32),
            pltpu.VMEM((K, RPS), jnp.float32),
            pltpu.VMEM((FCW, RPS), jnp.float32),
        ],
    )(feat_t, wt, nt)


def kernel(features, sel_idx_up, weights_down, nidx_down):
    n_down = weights_down.shape[0]
    pad = N_PAD - n_down
    wt = jnp.pad(weights_down, ((0, pad), (0, 0))).T
    nt = jnp.pad(nidx_down, ((0, pad), (0, 0))).T
    out_t = _sc_call(features.T, wt, nt)
    return out_t.T[:n_down]
